# baseline (speedup 1.0000x reference)
import functools

import jax
import jax.numpy as jnp
from jax import lax
from jax.experimental import pallas as pl
from jax.experimental.pallas import tpu as pltpu
from jax.experimental.pallas import tpu_sc as plsc

NC = 2
NS = 16
NW = NC * NS
L = 16


@functools.lru_cache(maxsize=None)
def _make_kernel(B, S, V, D, C):
    s_per_w = S // NW
    chunks = s_per_w // C
    ncol = D // L

    mesh = plsc.VectorSubcoreMesh(core_axis_name="c", subcore_axis_name="s")

    @functools.partial(
        pl.kernel,
        mesh=mesh,
        out_type=jax.ShapeDtypeStruct((B * S, D), jnp.float32),
        scratch_types=[
            pltpu.VMEM((B, s_per_w), jnp.int32),
            pltpu.VMEM((B, C, D), jnp.float32),
            pltpu.VMEM((C, D), jnp.float32),
            pltpu.SemaphoreType.DMA((B,)),
            pltpu.SemaphoreType.DMA((B,)),
            pltpu.SemaphoreType.DMA,
        ],
    )
    def emb_kernel(ids_hbm, tok_hbm, pos_hbm, out_hbm, idx_v, tokb, posb, gsem, ssem, psem):
        wid = lax.axis_index("s") * NC + lax.axis_index("c")
        s0 = wid * s_per_w

        for b in range(B):
            pltpu.sync_copy(ids_hbm.at[pl.ds(b * S + s0, s_per_w)], idx_v.at[b])

        def gather(k, b):
            pltpu.async_copy(
                tok_hbm.at[idx_v.at[b, pl.ds(k * C, C)]], tokb.at[b], gsem.at[b]
            )

        def gather_wait(b):
            pltpu.make_async_copy(
                tok_hbm.at[pl.ds(0, C)], tokb.at[b], gsem.at[b]
            ).wait()

        def scatter(k, b):
            pltpu.async_copy(
                tokb.at[b], out_hbm.at[pl.ds(b * S + s0 + k * C, C)], ssem.at[b]
            )

        def scatter_wait(b):
            pltpu.make_async_copy(
                tokb.at[b], out_hbm.at[pl.ds(b * S + s0, C)], ssem.at[b]
            ).wait()

        def pos_load(k):
            pltpu.async_copy(pos_hbm.at[pl.ds(s0 + k * C, C)], posb, psem)

        def pos_wait():
            pltpu.make_async_copy(pos_hbm.at[pl.ds(s0, C)], posb, psem).wait()

        pos_load(0)
        gather(0, 0)
        gather(0, 1)

        def outer(i, carry):
            for kk in range(2):
                k = i * 2 + kk

                for b in range(B):
                    if b == 0:
                        pos_wait()
                    gather_wait(b)

                    b2 = (b + 2) % B
                    k2 = k + (b + 2) // B
                    if b < 2:
                        if kk == 0:
                            @pl.when(i >= 1)
                            def _():
                                scatter_wait(b2)
                        else:
                            scatter_wait(b2)
                        gather(k2, b2)
                    else:
                        if kk == 0:
                            scatter_wait(b2)
                            gather(k2, b2)
                        else:
                            @pl.when(i < chunks // 2 - 1)
                            def _():
                                scatter_wait(b2)
                                gather(k2, b2)

                    def row_body(r, c2):
                        for c in range(ncol):
                            sl = pl.ds(c * L, L)
                            tokb[b, r, sl] = tokb[b, r, sl] + posb[r, sl]
                        return c2

                    lax.fori_loop(0, C, row_body, 0)

                    if b == B - 1:
                        if kk == 0:
                            pos_load(k + 1)
                        else:
                            @pl.when(i < chunks // 2 - 1)
                            def _():
                                pos_load(k + 1)

                    scatter(k, b)
            return carry

        lax.fori_loop(0, chunks // 2, outer, 0)

        for b in range(B):
            scatter_wait(b)

    return emb_kernel


def kernel(input_ids, token_embeddings, position_embeddings):
    B, S = input_ids.shape
    V, D = token_embeddings.shape
    ids = input_ids.reshape(-1).astype(jnp.int32)
    k = _make_kernel(B, S, V, D, 32)
    out = k(ids, token_embeddings, position_embeddings)
    return out.reshape(B, S, D)


# speedup vs baseline: 1.0303x; 1.0303x over previous
import functools

import jax
import jax.numpy as jnp
from jax import lax
from jax.experimental import pallas as pl
from jax.experimental.pallas import tpu as pltpu
from jax.experimental.pallas import tpu_sc as plsc

NC = 2
NS = 16
NW = NC * NS
L = 16


@functools.lru_cache(maxsize=None)
def _make_kernel(B, S, V, D, C):
    s_per_w = S // NW
    chunks = s_per_w // C
    ncol = D // L

    mesh = plsc.VectorSubcoreMesh(core_axis_name="c", subcore_axis_name="s")

    @functools.partial(
        pl.kernel,
        mesh=mesh,
        out_type=jax.ShapeDtypeStruct((B * S, D), jnp.float32),
        scratch_types=[
            pltpu.VMEM((B, s_per_w), jnp.int32),
            pltpu.VMEM((B, C, D), jnp.float32),
            pltpu.VMEM((B, C, D), jnp.float32),
            pltpu.VMEM((2, C, D), jnp.float32),
            pltpu.SemaphoreType.DMA((B,)),
            pltpu.SemaphoreType.DMA((B,)),
            pltpu.SemaphoreType.DMA((2,)),
        ],
    )
    def emb_kernel(ids_hbm, tok_hbm, pos_hbm, out_hbm, idx_v, tokb, stage, posb, gsem, ssem, psem):
        wid = lax.axis_index("s") * NC + lax.axis_index("c")
        s0 = wid * s_per_w

        for b in range(B):
            pltpu.sync_copy(ids_hbm.at[pl.ds(b * S + s0, s_per_w)], idx_v.at[b])

        def gather(k, b):
            pltpu.async_copy(
                tok_hbm.at[idx_v.at[b, pl.ds(k * C, C)]], tokb.at[b], gsem.at[b]
            )

        def gather_wait(b):
            pltpu.make_async_copy(
                tok_hbm.at[pl.ds(0, C)], tokb.at[b], gsem.at[b]
            ).wait()

        def scatter(k, b):
            pltpu.async_copy(
                stage.at[b], out_hbm.at[pl.ds(b * S + s0 + k * C, C)], ssem.at[b]
            )

        def scatter_wait(b):
            pltpu.make_async_copy(
                stage.at[b], out_hbm.at[pl.ds(0, C)], ssem.at[b]
            ).wait()

        def pos_load(k, pb):
            pltpu.async_copy(pos_hbm.at[pl.ds(s0 + k * C, C)], posb.at[pb], psem.at[pb])

        def pos_wait(pb):
            pltpu.make_async_copy(
                pos_hbm.at[pl.ds(s0, C)], posb.at[pb], psem.at[pb]
            ).wait()

        pos_load(0, 0)
        gather(0, 0)
        gather(0, 1)

        def outer(i, carry):
            for kk in range(2):
                k = i * 2 + kk
                pb = kk
                pos_wait(pb)
                if kk == 0:
                    pos_load(k + 1, 1 - pb)
                else:
                    @pl.when(i < chunks // 2 - 1)
                    def _():
                        pos_load(k + 1, 1 - pb)

                for b in range(B):
                    gather_wait(b)

                    b2 = (b + 2) % B
                    k2 = k + (b + 2) // B
                    if b < 2:
                        gather(k2, b2)
                    else:
                        if kk == 0:
                            gather(k2, b2)
                        else:
                            @pl.when(i < chunks // 2 - 1)
                            def _():
                                gather(k2, b2)

                    if kk == 0:
                        @pl.when(i >= 1)
                        def _():
                            scatter_wait(b)
                    else:
                        scatter_wait(b)

                    def row_body(r, c2):
                        for c in range(ncol):
                            sl = pl.ds(c * L, L)
                            stage[b, r, sl] = tokb[b, r, sl] + posb[pb, r, sl]
                        return c2

                    lax.fori_loop(0, C, row_body, 0)
                    scatter(k, b)
            return carry

        lax.fori_loop(0, chunks // 2, outer, 0)

        for b in range(B):
            scatter_wait(b)

    return emb_kernel


def kernel(input_ids, token_embeddings, position_embeddings):
    B, S = input_ids.shape
    V, D = token_embeddings.shape
    ids = input_ids.reshape(-1).astype(jnp.int32)
    k = _make_kernel(B, S, V, D, 16)
    out = k(ids, token_embeddings, position_embeddings)
    return out.reshape(B, S, D)


# speedup vs baseline: 2.0957x; 2.0341x over previous
import functools

import jax
import jax.numpy as jnp
from jax import lax
from jax.experimental import pallas as pl
from jax.experimental.pallas import tpu as pltpu
from jax.experimental.pallas import tpu_sc as plsc

NC = 2
NS = 16
NW = NC * NS
L = 16


@functools.lru_cache(maxsize=None)
def _make_kernel(B, S, V, D, C):
    s_per_w = S // NW
    chunks = s_per_w // C
    ncol = D // L
    nsteps = chunks * B

    mesh = plsc.VectorSubcoreMesh(core_axis_name="c", subcore_axis_name="s")

    @functools.partial(
        pl.kernel,
        mesh=mesh,
        out_type=jax.ShapeDtypeStruct((B * S, D), jnp.float32),
        scratch_types=[
            pltpu.VMEM((B, s_per_w), jnp.int32),
            pltpu.VMEM((2 * B, C, D), jnp.float32),
            pltpu.VMEM((2, C, D), jnp.float32),
            pltpu.SemaphoreType.DMA((2 * B,)),
            pltpu.SemaphoreType.DMA((2 * B,)),
            pltpu.SemaphoreType.DMA((2,)),
        ],
    )
    def emb_kernel(ids_hbm, tok_hbm, pos_hbm, out_hbm, idx_v, tokb, posb, gsem, ssem, psem):
        wid = lax.axis_index("s") * NC + lax.axis_index("c")
        s0 = wid * s_per_w

        for b in range(B):
            pltpu.sync_copy(ids_hbm.at[pl.ds(b * S + s0, s_per_w)], idx_v.at[b])

        def gather(k, b, u):
            pltpu.async_copy(
                tok_hbm.at[idx_v.at[b, pl.ds(k * C, C)]], tokb.at[u], gsem.at[u]
            )

        def gather_wait(u):
            pltpu.make_async_copy(
                tok_hbm.at[pl.ds(0, C)], tokb.at[u], gsem.at[u]
            ).wait()

        def scatter(k, b, u):
            pltpu.async_copy(
                tokb.at[u], out_hbm.at[pl.ds(b * S + s0 + k * C, C)], ssem.at[u]
            )

        def scatter_wait(u):
            pltpu.make_async_copy(
                tokb.at[u], out_hbm.at[pl.ds(0, C)], ssem.at[u]
            ).wait()

        def pos_load(k, pb):
            pltpu.async_copy(pos_hbm.at[pl.ds(s0 + k * C, C)], posb.at[pb], psem.at[pb])

        def pos_wait(pb):
            pltpu.make_async_copy(
                pos_hbm.at[pl.ds(s0, C)], posb.at[pb], psem.at[pb]
            ).wait()

        pos_load(0, 0)
        for b in range(B):
            gather(0, b, b)

        def outer(j, carry):
            for ss in range(2 * B):
                k = 2 * j + ss // B
                b = ss % B
                pb = ss // B
                u2 = (ss + B) % (2 * B)

                if b == 0:
                    pos_wait(pb)
                    if ss == 0:
                        pos_load(k + 1, 1 - pb)
                    else:
                        @pl.when(j < chunks // 2 - 1)
                        def _():
                            pos_load(k + 1, 1 - pb)

                gather_wait(ss)

                if ss < B:
                    @pl.when(j >= 1)
                    def _():
                        scatter_wait(u2)
                    gather(k + 1, b, u2)
                else:
                    @pl.when(j < chunks // 2 - 1)
                    def _():
                        scatter_wait(u2)
                        gather(k + 1, b, u2)

                def row_body(r, c2):
                    for c in range(ncol):
                        sl = pl.ds(c * L, L)
                        tokb[ss, r, sl] = tokb[ss, r, sl] + posb[pb, r, sl]
                    return c2

                lax.fori_loop(0, C, row_body, 0)
                scatter(k, b, ss)
            return carry

        lax.fori_loop(0, chunks // 2, outer, 0)

        for u in range(2 * B):
            scatter_wait(u)

    return emb_kernel


def kernel(input_ids, token_embeddings, position_embeddings):
    B, S = input_ids.shape
    V, D = token_embeddings.shape
    ids = input_ids.reshape(-1).astype(jnp.int32)
    k = _make_kernel(B, S, V, D, 8)
    out = k(ids, token_embeddings, position_embeddings)
    return out.reshape(B, S, D)


# speedup vs baseline: 2.1757x; 1.0381x over previous
import functools

import jax
import jax.numpy as jnp
from jax import lax
from jax.experimental import pallas as pl
from jax.experimental.pallas import tpu as pltpu
from jax.experimental.pallas import tpu_sc as plsc

NC = 2
NS = 16
NW = NC * NS
L = 16


@functools.lru_cache(maxsize=None)
def _make_kernel(B, S, V, D, C):
    s_per_w = S // NW
    chunks = s_per_w // C
    ncol = D // L
    nsteps = chunks * B

    mesh = plsc.VectorSubcoreMesh(core_axis_name="c", subcore_axis_name="s")

    @functools.partial(
        pl.kernel,
        mesh=mesh,
        out_type=jax.ShapeDtypeStruct((B * S, D), jnp.float32),
        scratch_types=[
            pltpu.VMEM((B, s_per_w), jnp.int32),
            pltpu.VMEM((2 * B, C, D), jnp.float32),
            pltpu.VMEM((2, C, D), jnp.float32),
            pltpu.SemaphoreType.DMA((2 * B,)),
            pltpu.SemaphoreType.DMA((2 * B,)),
            pltpu.SemaphoreType.DMA((2,)),
        ],
    )
    def emb_kernel(ids_hbm, tok_hbm, pos_hbm, out_hbm, idx_v, tokb, posb, gsem, ssem, psem):
        wid = lax.axis_index("s") * NC + lax.axis_index("c")
        s0 = wid * s_per_w

        for b in range(B):
            pltpu.sync_copy(ids_hbm.at[pl.ds(b * S + s0, s_per_w)], idx_v.at[b])

        def gather(k, b, u):
            pltpu.async_copy(
                tok_hbm.at[idx_v.at[b, pl.ds(k * C, C)]], tokb.at[u], gsem.at[u]
            )

        def gather_wait(u):
            pltpu.make_async_copy(
                tok_hbm.at[pl.ds(0, C)], tokb.at[u], gsem.at[u]
            ).wait()

        def scatter(k, b, u):
            pltpu.async_copy(
                tokb.at[u], out_hbm.at[pl.ds(b * S + s0 + k * C, C)], ssem.at[u]
            )

        def scatter_wait(u):
            pltpu.make_async_copy(
                tokb.at[u], out_hbm.at[pl.ds(0, C)], ssem.at[u]
            ).wait()

        def pos_load(k, pb):
            pltpu.async_copy(pos_hbm.at[pl.ds(s0 + k * C, C)], posb.at[pb], psem.at[pb])

        def pos_wait(pb):
            pltpu.make_async_copy(
                pos_hbm.at[pl.ds(s0, C)], posb.at[pb], psem.at[pb]
            ).wait()

        pos_load(0, 0)
        for b in range(B):
            gather(0, b, b)

        def outer(j, carry):
            for ss in range(0, 2 * B, 2):
                k = 2 * j + ss // B
                b = ss % B
                pb = ss // B

                if b == 0:
                    pos_wait(pb)
                    if ss == 0:
                        pos_load(k + 1, 1 - pb)
                    else:
                        @pl.when(j < chunks // 2 - 1)
                        def _():
                            pos_load(k + 1, 1 - pb)

                gather_wait(ss)
                gather_wait(ss + 1)

                for p in range(2):
                    u2 = (ss + p + B) % (2 * B)
                    if ss < B:
                        @pl.when(j >= 1)
                        def _():
                            scatter_wait(u2)
                        gather(k + 1, b + p, u2)
                    else:
                        @pl.when(j < chunks // 2 - 1)
                        def _():
                            scatter_wait(u2)
                            gather(k + 1, b + p, u2)

                def row_body(r, c2):
                    for c in range(ncol):
                        sl = pl.ds(c * L, L)
                        pv = posb[pb, r, sl]
                        for p in range(2):
                            tokb[ss + p, r, sl] = tokb[ss + p, r, sl] + pv
                    return c2

                lax.fori_loop(0, C, row_body, 0)
                scatter(k, b, ss)
                scatter(k, b + 1, ss + 1)
            return carry

        lax.fori_loop(0, chunks // 2, outer, 0)

        for u in range(2 * B):
            scatter_wait(u)

    return emb_kernel


def kernel(input_ids, token_embeddings, position_embeddings):
    B, S = input_ids.shape
    V, D = token_embeddings.shape
    ids = input_ids.reshape(-1).astype(jnp.int32)
    k = _make_kernel(B, S, V, D, 8)
    out = k(ids, token_embeddings, position_embeddings)
    return out.reshape(B, S, D)


# speedup vs baseline: 2.2113x; 1.0164x over previous
import functools

import jax
import jax.numpy as jnp
from jax import lax
from jax.experimental import pallas as pl
from jax.experimental.pallas import tpu as pltpu
from jax.experimental.pallas import tpu_sc as plsc

NC = 2
NS = 16
NW = NC * NS
L = 16


@functools.lru_cache(maxsize=None)
def _make_kernel(B, S, V, D, C):
    s_per_w = S // NW
    chunks = s_per_w // C
    ncol = D // L
    nsteps = chunks * B

    mesh = plsc.VectorSubcoreMesh(core_axis_name="c", subcore_axis_name="s")

    @functools.partial(
        pl.kernel,
        mesh=mesh,
        out_type=jax.ShapeDtypeStruct((B * S, D), jnp.float32),
        scratch_types=[
            pltpu.VMEM((B, s_per_w), jnp.int32),
            pltpu.VMEM((2 * B, C, D), jnp.float32),
            pltpu.VMEM((2, C, D), jnp.float32),
            pltpu.SemaphoreType.DMA((2 * B,)),
            pltpu.SemaphoreType.DMA((2 * B,)),
            pltpu.SemaphoreType.DMA((2,)),
        ],
    )
    def emb_kernel(ids_hbm, tok_hbm, pos_hbm, out_hbm, idx_v, tokb, posb, gsem, ssem, psem):
        wid = lax.axis_index("s") * NC + lax.axis_index("c")
        s0 = wid * s_per_w

        for b in range(B):
            pltpu.async_copy(ids_hbm.at[pl.ds(b * S + s0, s_per_w)], idx_v.at[b], ssem.at[b])
        for b in range(B):
            pltpu.make_async_copy(
                ids_hbm.at[pl.ds(b * S + s0, s_per_w)], idx_v.at[b], ssem.at[b]
            ).wait()

        def gather(k, b, u):
            pltpu.async_copy(
                tok_hbm.at[idx_v.at[b, pl.ds(k * C, C)]], tokb.at[u], gsem.at[u]
            )

        def gather_wait(u):
            pltpu.make_async_copy(
                tok_hbm.at[pl.ds(0, C)], tokb.at[u], gsem.at[u]
            ).wait()

        def scatter(k, b, u):
            pltpu.async_copy(
                tokb.at[u], out_hbm.at[pl.ds(b * S + s0 + k * C, C)], ssem.at[u]
            )

        def scatter_wait(u):
            pltpu.make_async_copy(
                tokb.at[u], out_hbm.at[pl.ds(0, C)], ssem.at[u]
            ).wait()

        def pos_load(k, pb):
            pltpu.async_copy(pos_hbm.at[pl.ds(s0 + k * C, C)], posb.at[pb], psem.at[pb])

        def pos_wait(pb):
            pltpu.make_async_copy(
                pos_hbm.at[pl.ds(s0, C)], posb.at[pb], psem.at[pb]
            ).wait()

        pos_load(0, 0)
        for b in range(B):
            gather(0, b, b)
        gather(1, 0, 4)
        gather(1, 1, 5)

        def outer(j, carry):
            for ss in range(0, 2 * B, 2):
                k = 2 * j + ss // B
                b = ss % B
                pb = ss // B

                if b == 0:
                    pos_wait(pb)
                    if ss == 0:
                        pos_load(k + 1, 1 - pb)
                    else:
                        @pl.when(j < chunks // 2 - 1)
                        def _():
                            pos_load(k + 1, 1 - pb)

                gather_wait(ss)
                gather_wait(ss + 1)

                kr = 2 * j + (ss + 6) // 4
                br = (ss + 6) % 4
                for p in range(2):
                    u2 = (ss + 6 + p) % (2 * B)
                    if ss == 0:
                        @pl.when(j >= 1)
                        def _():
                            scatter_wait(u2)
                        gather(kr, br + p, u2)
                    else:
                        @pl.when(j < chunks // 2 - 1)
                        def _():
                            scatter_wait(u2)
                            gather(kr, br + p, u2)

                def row_body(r, c2):
                    for c in range(ncol):
                        sl = pl.ds(c * L, L)
                        pv = posb[pb, r, sl]
                        for p in range(2):
                            tokb[ss + p, r, sl] = tokb[ss + p, r, sl] + pv
                    return c2

                lax.fori_loop(0, C, row_body, 0)
                scatter(k, b, ss)
                scatter(k, b + 1, ss + 1)
            return carry

        lax.fori_loop(0, chunks // 2, outer, 0)

        for u in range(2 * B):
            scatter_wait(u)

    return emb_kernel


def kernel(input_ids, token_embeddings, position_embeddings):
    B, S = input_ids.shape
    V, D = token_embeddings.shape
    ids = input_ids.reshape(-1).astype(jnp.int32)
    k = _make_kernel(B, S, V, D, 8)
    out = k(ids, token_embeddings, position_embeddings)
    return out.reshape(B, S, D)
